# Initial kernel scaffold; baseline (speedup 1.0000x reference)
import jax
import jax.numpy as jnp
from jax.experimental import pallas as pl


def kernel(scores):
    raise NotImplementedError("write your pallas kernel here")


# speedup vs baseline: 11.9624x; 11.9624x over previous
import functools

import jax
import jax.numpy as jnp
from jax import lax
from jax.experimental import pallas as pl
from jax.experimental.pallas import tpu as pltpu

K = 1024
ROWS_PER_BLOCK = 8


def _topk_mask_block(x_ref, o_ref, *, k: int):
    x = x_ref[...]
    u = lax.bitcast_convert_type(x, jnp.uint32)
    sign = u >> jnp.uint32(31)
    flip = (jnp.uint32(0) - sign) | jnp.uint32(0x80000000)
    ukey = u ^ flip

    def step(i, t):
        bit = jnp.uint32(1) << (jnp.uint32(31) - i.astype(jnp.uint32))
        cand = t | bit
        cnt = jnp.sum((ukey >= cand).astype(jnp.int32), axis=1, keepdims=True)
        return jnp.where(cnt >= k, cand, t)

    t0 = jnp.zeros((x.shape[0], 1), jnp.uint32)
    t = lax.fori_loop(0, 32, step, t0)
    o_ref[...] = (ukey >= t).astype(jnp.float32)


def kernel(scores):
    b, n = scores.shape
    k = min(K, n)
    grid = (b // ROWS_PER_BLOCK,)
    return pl.pallas_call(
        functools.partial(_topk_mask_block, k=k),
        grid=grid,
        in_specs=[pl.BlockSpec((ROWS_PER_BLOCK, n), lambda i: (i, 0))],
        out_specs=pl.BlockSpec((ROWS_PER_BLOCK, n), lambda i: (i, 0)),
        out_shape=jax.ShapeDtypeStruct((b, n), jnp.float32),
    )(scores)
